# Initial kernel scaffold; baseline (speedup 1.0000x reference)
import jax
import jax.numpy as jnp
from jax.experimental import pallas as pl


def kernel(oldx, W_in, b_in, W_out):
    raise NotImplementedError("write your pallas kernel here")


# speedup vs baseline: 2.4947x; 2.4947x over previous
import functools

import jax
import jax.numpy as jnp
from jax.experimental import pallas as pl
from jax.experimental.pallas import tpu as pltpu

IN_W = 2048
OUT_W = 2048
DEPTH = 3
P = 8
FANOUT = 4
G = 85
N_HEAD_G = 21
GPAD = 32
HEAD_W = FANOUT * P * GPAD
REST_W = P * (G - N_HEAD_G) * FANOUT
TOT_W = HEAD_W + REST_W

TB = 256


def _gelu_exact(x):
    return 0.5 * x * (1.0 + jax.lax.erf(x * (2.0 ** -0.5)))


def _ffff_body(x_ref, w1h_ref, w1r_ref, b1_ref, w2_ref, o_ref):
    x = x_ref[...]
    x_bf = x.astype(jnp.bfloat16)

    z_head = jax.lax.dot_general(
        x, w1h_ref[...], (((1,), (1,)), ((), ())),
        preferred_element_type=jnp.float32)
    z_rest = jax.lax.dot_general(
        x_bf, w1r_ref[...], (((1,), (1,)), ((), ())),
        preferred_element_type=jnp.float32)
    b1 = b1_ref[...]
    a_head = _gelu_exact(z_head + b1[:, :HEAD_W])
    a_rest = _gelu_exact(z_rest + b1[:, HEAD_W:])

    a0 = a_head[:, 0 * 256:1 * 256]
    a1 = a_head[:, 1 * 256:2 * 256]
    a2 = a_head[:, 2 * 256:3 * 256]
    a3 = a_head[:, 3 * 256:4 * 256]
    dec = jnp.where(a1 > a0, 1, 0)
    m = jnp.maximum(a0, a1)
    dec = jnp.where(a2 > m, 2, dec)
    m = jnp.maximum(m, a2)
    dec = jnp.where(a3 > m, 3, dec)

    iota32 = jax.lax.broadcasted_iota(jnp.int32, (1, GPAD), 1)
    g1s, g2s, g3s = [], [], []
    for p in range(P):
        dec_p = dec[:, p * GPAD:(p + 1) * GPAD]
        mv0 = dec_p[:, 0:1]
        g1 = 1 + mv0
        oh1 = (iota32 == g1).astype(jnp.int32)
        mv1 = jnp.sum(dec_p * oh1, axis=1, keepdims=True)
        g2 = 5 + 4 * (g1 - 1) + mv1
        oh2 = (iota32 == g2).astype(jnp.int32)
        mv2 = jnp.sum(dec_p * oh2, axis=1, keepdims=True)
        g3 = 21 + 4 * (g2 - 5) + mv2
        g1s.append(g1)
        g2s.append(g2)
        g3s.append(g3)

    hio = jax.lax.broadcasted_iota(jnp.int32, (1, HEAD_W), 1)
    h_g = jax.lax.rem(hio, GPAD)
    h_p = jax.lax.rem(jax.lax.div(hio, GPAD), P)
    g1f = jnp.zeros((x.shape[0], HEAD_W), jnp.int32)
    g2f = jnp.zeros((x.shape[0], HEAD_W), jnp.int32)
    for p in range(P):
        sel = h_p == p
        g1f = jnp.where(sel, g1s[p], g1f)
        g2f = jnp.where(sel, g2s[p], g2f)
    hmask = (h_g == 0) | (h_g == g1f) | (h_g == g2f)

    rio = jax.lax.broadcasted_iota(jnp.int32, (1, REST_W), 1)
    r_p = jax.lax.div(rio, 256)
    r_g = 21 + jax.lax.div(jax.lax.rem(rio, 256), FANOUT)
    g3f = jnp.zeros((x.shape[0], REST_W), jnp.int32)
    for p in range(P):
        g3f = jnp.where(r_p == p, g3s[p], g3f)
    rmask = r_g == g3f

    am = jnp.concatenate(
        [jnp.where(hmask, a_head, 0.0), jnp.where(rmask, a_rest, 0.0)],
        axis=1).astype(jnp.bfloat16)
    o_ref[...] = jax.lax.dot_general(
        am, w2_ref[...], (((1,), (1,)), ((), ())),
        preferred_element_type=jnp.float32)


@functools.partial(jax.jit, static_argnames=())
def _ffff(x, W1h, W1r, b1, W2):
    B = x.shape[0]
    grid = (B // TB,)
    return pl.pallas_call(
        _ffff_body,
        grid=grid,
        in_specs=[
            pl.BlockSpec((TB, IN_W), lambda i: (i, 0)),
            pl.BlockSpec((HEAD_W, IN_W), lambda i: (0, 0)),
            pl.BlockSpec((REST_W, IN_W), lambda i: (0, 0)),
            pl.BlockSpec((1, TOT_W), lambda i: (0, 0)),
            pl.BlockSpec((OUT_W, TOT_W), lambda i: (0, 0)),
        ],
        out_specs=pl.BlockSpec((TB, OUT_W), lambda i: (i, 0)),
        out_shape=jax.ShapeDtypeStruct((B, OUT_W), jnp.float32),
        compiler_params=pltpu.CompilerParams(
            dimension_semantics=("parallel",)),
    )(x, W1h, W1r, b1, W2)


def kernel(oldx, W_in, b_in, W_out):
    x = oldx.reshape(-1, IN_W)

    Wi = W_in.reshape(P, G, FANOUT, IN_W)
    bi = b_in.reshape(P, G, FANOUT)
    Wh = jnp.transpose(Wi[:, :N_HEAD_G], (2, 0, 1, 3))
    Wh = jnp.pad(Wh, ((0, 0), (0, 0), (0, GPAD - N_HEAD_G), (0, 0)))
    W1h = Wh.reshape(HEAD_W, IN_W)
    bh = jnp.transpose(bi[:, :N_HEAD_G], (2, 0, 1))
    bh = jnp.pad(bh, ((0, 0), (0, 0), (0, GPAD - N_HEAD_G)))
    b1h = bh.reshape(HEAD_W)
    W1r = Wi[:, N_HEAD_G:].reshape(REST_W, IN_W).astype(jnp.bfloat16)
    b1r = bi[:, N_HEAD_G:].reshape(REST_W)
    b1 = jnp.concatenate([b1h, b1r]).reshape(1, TOT_W)

    Wo = W_out.reshape(OUT_W, P, G, FANOUT)
    Woh = jnp.transpose(Wo[:, :, :N_HEAD_G], (0, 3, 1, 2))
    Woh = jnp.pad(Woh, ((0, 0), (0, 0), (0, 0), (0, GPAD - N_HEAD_G)))
    W2 = jnp.concatenate(
        [Woh.reshape(OUT_W, HEAD_W),
         Wo[:, :, N_HEAD_G:].reshape(OUT_W, REST_W)],
        axis=1).astype(jnp.bfloat16)

    out = _ffff(x, W1h, W1r, b1, W2)
    return out.reshape(oldx.shape)


# speedup vs baseline: 3.2126x; 1.2878x over previous
import functools

import jax
import jax.numpy as jnp
from jax.experimental import pallas as pl
from jax.experimental.pallas import tpu as pltpu

IN_W = 2048
OUT_W = 2048
P = 8
FANOUT = 4
G = 85
N_HEAD_G = 21
GPAD = 32
SEG = P * GPAD
HEAD_W = FANOUT * SEG
REST_W = P * (G - N_HEAD_G) * FANOUT
TOT_W = HEAD_W + REST_W

TB = 256


def _gelu_exact(x):
    return 0.5 * x * (1.0 + jax.lax.erf(x * (2.0 ** -0.5)))


def _ffff_body(x_ref, w1h_ref, w1r_ref, b1_ref, w2_ref, sseg_ref, sexp_ref,
               o_ref):
    x = x_ref[...]
    x_bf = x.astype(jnp.bfloat16)

    z_head = jax.lax.dot_general(
        x, w1h_ref[...], (((1,), (1,)), ((), ())),
        preferred_element_type=jnp.float32)
    z_rest = jax.lax.dot_general(
        x_bf, w1r_ref[...], (((1,), (1,)), ((), ())),
        preferred_element_type=jnp.float32)
    b1 = b1_ref[...]
    a_head = _gelu_exact(z_head + b1[:, :HEAD_W])
    a_rest = _gelu_exact(z_rest + b1[:, HEAD_W:])

    a0 = a_head[:, 0 * SEG:1 * SEG]
    a1 = a_head[:, 1 * SEG:2 * SEG]
    a2 = a_head[:, 2 * SEG:3 * SEG]
    a3 = a_head[:, 3 * SEG:4 * SEG]
    one = jnp.float32(1.0)
    dec = jnp.where(a1 > a0, one, 0.0)
    m = jnp.maximum(a0, a1)
    dec = jnp.where(a2 > m, 2.0, dec)
    m = jnp.maximum(m, a2)
    dec = jnp.where(a3 > m, 3.0, dec)

    sseg = sseg_ref[...]
    sexp = sexp_ref[...]

    def segb(v):
        return jax.lax.dot_general(
            v.astype(jnp.bfloat16), sseg, (((1,), (0,)), ((), ())),
            preferred_element_type=jnp.float32)

    gio = jax.lax.rem(
        jax.lax.broadcasted_iota(jnp.int32, (1, SEG), 1), GPAD
    ).astype(jnp.float32)

    oh0 = jnp.where(gio == 0.0, one, 0.0)
    g1 = 1.0 + segb(dec * oh0)
    oh1 = jnp.where(gio == g1, one, 0.0)
    g2 = 1.0 + 4.0 * g1 + segb(dec * oh1)
    oh2 = jnp.where(gio == g2, one, 0.0)
    g3 = 1.0 + 4.0 * g2 + segb(dec * oh2)

    hmask = (oh0 + oh1 + oh2).astype(jnp.bfloat16)
    hm = [(af * hmask.astype(jnp.float32)).astype(jnp.bfloat16)
          for af in (a0, a1, a2, a3)]

    g3e = jax.lax.dot_general(
        g3.astype(jnp.bfloat16), sexp, (((1,), (0,)), ((), ())),
        preferred_element_type=jnp.float32)
    rio = jax.lax.broadcasted_iota(jnp.int32, (1, REST_W), 1)
    r_g = (21 + jax.lax.div(jax.lax.rem(rio, 256), FANOUT)).astype(
        jnp.float32)
    am_rest = jnp.where(r_g == g3e, a_rest, 0.0).astype(jnp.bfloat16)

    am = jnp.concatenate(hm + [am_rest], axis=1)
    o_ref[...] = jax.lax.dot_general(
        am, w2_ref[...], (((1,), (1,)), ((), ())),
        preferred_element_type=jnp.float32)


@jax.jit
def _ffff(x, W1h, W1r, b1, W2, Sseg, Sexp):
    B = x.shape[0]
    grid = (B // TB,)
    return pl.pallas_call(
        _ffff_body,
        grid=grid,
        in_specs=[
            pl.BlockSpec((TB, IN_W), lambda i: (i, 0)),
            pl.BlockSpec((HEAD_W, IN_W), lambda i: (0, 0)),
            pl.BlockSpec((REST_W, IN_W), lambda i: (0, 0)),
            pl.BlockSpec((1, TOT_W), lambda i: (0, 0)),
            pl.BlockSpec((OUT_W, TOT_W), lambda i: (0, 0)),
            pl.BlockSpec((SEG, SEG), lambda i: (0, 0)),
            pl.BlockSpec((SEG, REST_W), lambda i: (0, 0)),
        ],
        out_specs=pl.BlockSpec((TB, OUT_W), lambda i: (i, 0)),
        out_shape=jax.ShapeDtypeStruct((B, OUT_W), jnp.float32),
        compiler_params=pltpu.CompilerParams(
            dimension_semantics=("parallel",)),
    )(x, W1h, W1r, b1, W2, Sseg, Sexp)


def kernel(oldx, W_in, b_in, W_out):
    x = oldx.reshape(-1, IN_W)

    Wi = W_in.reshape(P, G, FANOUT, IN_W)
    bi = b_in.reshape(P, G, FANOUT)
    Wh = jnp.transpose(Wi[:, :N_HEAD_G], (2, 0, 1, 3))
    Wh = jnp.pad(Wh, ((0, 0), (0, 0), (0, GPAD - N_HEAD_G), (0, 0)))
    W1h = Wh.reshape(HEAD_W, IN_W)
    bh = jnp.transpose(bi[:, :N_HEAD_G], (2, 0, 1))
    bh = jnp.pad(bh, ((0, 0), (0, 0), (0, GPAD - N_HEAD_G)))
    b1h = bh.reshape(HEAD_W)
    W1r = Wi[:, N_HEAD_G:].reshape(REST_W, IN_W).astype(jnp.bfloat16)
    b1r = bi[:, N_HEAD_G:].reshape(REST_W)
    b1 = jnp.concatenate([b1h, b1r]).reshape(1, TOT_W)

    Wo = W_out.reshape(OUT_W, P, G, FANOUT)
    Woh = jnp.transpose(Wo[:, :, :N_HEAD_G], (0, 3, 1, 2))
    Woh = jnp.pad(Woh, ((0, 0), (0, 0), (0, 0), (0, GPAD - N_HEAD_G)))
    W2 = jnp.concatenate(
        [Woh.reshape(OUT_W, HEAD_W),
         Wo[:, :, N_HEAD_G:].reshape(OUT_W, REST_W)],
        axis=1).astype(jnp.bfloat16)

    iseg = jnp.arange(SEG)
    Sseg = (iseg[:, None] // GPAD == iseg[None, :] // GPAD).astype(
        jnp.bfloat16)
    irest = jnp.arange(REST_W)
    Sexp = ((iseg[:, None] == (irest[None, :] // 256) * GPAD)).astype(
        jnp.bfloat16)

    out = _ffff(x, W1h, W1r, b1, W2, Sseg, Sexp)
    return out.reshape(oldx.shape)
